# baseline (speedup 1.0000x reference)
import functools

import jax
import jax.numpy as jnp
from jax import lax
from jax.experimental import pallas as pl
from jax.experimental.pallas import tpu as pltpu
from jax.experimental.pallas import tpu_sc as plsc

NUM_WORKERS = 32
CHUNK = 128
NBUF = 4
LANES = 16


def _pos_embed_body(x_hbm, idx_hbm, tab_hbm, out_hbm, tb, idx_v, xb, xsem, osem):
    nc = 2
    wid = lax.axis_index("s") * nc + lax.axis_index("c")
    tok_per_worker = x_hbm.shape[0] // NUM_WORKERS
    n_chunks = tok_per_worker // CHUNK
    worker_base = wid * tok_per_worker
    chunk_base = wid * n_chunks

    pltpu.sync_copy(tab_hbm, tb)
    pltpu.sync_copy(idx_hbm.at[pl.ds(chunk_base, n_chunks)], idx_v)

    viota = lax.iota(jnp.int32, LANES)

    def issue_in(g):
        b = lax.rem(g, NBUF)
        base = worker_base + g * CHUNK
        pltpu.async_copy(x_hbm.at[pl.ds(base, CHUNK)], xb.at[b], xsem.at[b])

    def compute_and_out(g):
        b = lax.rem(g, NBUF)
        base = worker_base + g * CHUNK
        pltpu.make_async_copy(
            x_hbm.at[pl.ds(base, CHUNK)], xb.at[b], xsem.at[b]
        ).wait()

        @pl.loop(0, CHUNK // LANES)
        def _grp(grp):
            tokv = grp * LANES + viota
            iv = idx_v[g, pl.ds(grp * LANES, LANES)]
            srcv = lax.shift_left(iv, jnp.int32(5))
            colv = viota * 0
            colv2 = colv + jnp.int32(32)
            for k in range(32):
                w = plsc.load_gather(tb, [srcv])
                plsc.addupdate_scatter(
                    xb.at[b],
                    [tokv, colv],
                    lax.bitcast_convert_type(
                        lax.shift_left(w, jnp.int32(16)), jnp.float32
                    ),
                )
                plsc.addupdate_scatter(
                    xb.at[b],
                    [tokv, colv2],
                    lax.bitcast_convert_type(
                        lax.bitwise_and(w, jnp.int32(-65536)), jnp.float32
                    ),
                )
                if k != 31:
                    srcv = srcv + jnp.int32(1)
                    colv = colv + jnp.int32(1)
                    colv2 = colv2 + jnp.int32(1)

        pltpu.async_copy(xb.at[b], out_hbm.at[pl.ds(base, CHUNK)], osem.at[b])

    def wait_out(g):
        b = lax.rem(g, NBUF)
        base = worker_base + g * CHUNK
        pltpu.make_async_copy(
            xb.at[b], out_hbm.at[pl.ds(base, CHUNK)], osem.at[b]
        ).wait()

    @pl.loop(0, n_chunks + 1)
    def _step(t):
        @pl.when(jnp.logical_and(t >= NBUF, t - NBUF < n_chunks))
        def _():
            wait_out(t - NBUF)

        @pl.when(t < n_chunks)
        def _():
            issue_in(t)

        @pl.when(t >= 1)
        def _():
            compute_and_out(t - 1)

    @pl.loop(max(0, n_chunks + 1 - NBUF), n_chunks)
    def _drain(g):
        wait_out(g)


@functools.partial(jax.jit, static_argnames=())
def kernel(x, position_indices, embedding):
    b, s, d = x.shape
    n = b * s
    x_flat = x.reshape(n, d)
    idx_2d = position_indices.reshape(n // CHUNK, CHUNK).astype(jnp.int32)

    eb = lax.bitcast_convert_type(
        embedding.astype(jnp.bfloat16), jnp.uint16
    ).astype(jnp.uint32)
    half = d // 2
    packed = jnp.bitwise_or(
        eb[:, :half], jnp.left_shift(eb[:, half:], jnp.uint32(16))
    ).astype(jnp.int32)
    tab_flat = packed.reshape(-1)

    mesh = plsc.VectorSubcoreMesh(
        core_axis_name="c", subcore_axis_name="s", num_cores=2, num_subcores=16
    )
    n_chunks_w = n // NUM_WORKERS // CHUNK
    out = pl.kernel(
        _pos_embed_body,
        out_type=jax.ShapeDtypeStruct((n, d), x.dtype),
        mesh=mesh,
        scratch_types=[
            pltpu.VMEM(tab_flat.shape, jnp.int32),
            pltpu.VMEM((n_chunks_w, CHUNK), jnp.int32),
            pltpu.VMEM((NBUF, CHUNK, d), jnp.float32),
            pltpu.SemaphoreType.DMA((NBUF,)),
            pltpu.SemaphoreType.DMA((NBUF,)),
        ],
        compiler_params=pltpu.CompilerParams(
            use_tc_tiling_on_sc=False, needs_layout_passes=False
        ),
    )(x_flat, idx_2d, tab_flat)
    return out.reshape(b, s, d)


# speedup vs baseline: 1.1209x; 1.1209x over previous
import functools

import jax
import jax.numpy as jnp
from jax import lax
from jax.experimental import pallas as pl
from jax.experimental.pallas import tpu as pltpu
from jax.experimental.pallas import tpu_sc as plsc

NUM_WORKERS = 32
CHUNK = 128
NBUF = 4
LANES = 16


def _pos_embed_body(x_hbm, idx_hbm, tab_hbm, out_hbm, tb, idx_v, xb, xsem, osem):
    nc = 2
    wid = lax.axis_index("s") * nc + lax.axis_index("c")
    tok_per_worker = x_hbm.shape[0] // NUM_WORKERS
    n_chunks = tok_per_worker // CHUNK
    worker_base = wid * tok_per_worker
    chunk_base = wid * n_chunks

    pltpu.sync_copy(tab_hbm, tb)
    pltpu.sync_copy(idx_hbm.at[pl.ds(chunk_base, n_chunks)], idx_v)

    viota = lax.iota(jnp.int32, LANES)

    def issue_in(g):
        b = lax.rem(g, NBUF)
        base = worker_base + g * CHUNK
        pltpu.async_copy(x_hbm.at[pl.ds(base, CHUNK)], xb.at[b], xsem.at[b])

    def compute_and_out(g):
        b = lax.rem(g, NBUF)
        base = worker_base + g * CHUNK
        pltpu.make_async_copy(
            x_hbm.at[pl.ds(base, CHUNK)], xb.at[b], xsem.at[b]
        ).wait()

        @plsc.parallel_loop(0, CHUNK // LANES)
        def _grp(grp):
            tokv = grp * LANES + viota
            iv = idx_v[g, pl.ds(grp * LANES, LANES)]
            srcv = lax.shift_left(iv, jnp.int32(5))
            colv = viota * 0
            colv2 = colv + jnp.int32(32)
            for k in range(32):
                w = plsc.load_gather(tb, [srcv])
                plsc.addupdate_scatter(
                    xb.at[b],
                    [tokv, colv],
                    lax.bitcast_convert_type(
                        lax.shift_left(w, jnp.int32(16)), jnp.float32
                    ),
                )
                plsc.addupdate_scatter(
                    xb.at[b],
                    [tokv, colv2],
                    lax.bitcast_convert_type(
                        lax.bitwise_and(w, jnp.int32(-65536)), jnp.float32
                    ),
                )
                if k != 31:
                    srcv = srcv + jnp.int32(1)
                    colv = colv + jnp.int32(1)
                    colv2 = colv2 + jnp.int32(1)

        pltpu.async_copy(xb.at[b], out_hbm.at[pl.ds(base, CHUNK)], osem.at[b])

    def wait_out(g):
        b = lax.rem(g, NBUF)
        base = worker_base + g * CHUNK
        pltpu.make_async_copy(
            xb.at[b], out_hbm.at[pl.ds(base, CHUNK)], osem.at[b]
        ).wait()

    @pl.loop(0, n_chunks + 1)
    def _step(t):
        @pl.when(jnp.logical_and(t >= NBUF, t - NBUF < n_chunks))
        def _():
            wait_out(t - NBUF)

        @pl.when(t < n_chunks)
        def _():
            issue_in(t)

        @pl.when(t >= 1)
        def _():
            compute_and_out(t - 1)

    @pl.loop(max(0, n_chunks + 1 - NBUF), n_chunks)
    def _drain(g):
        wait_out(g)


@functools.partial(jax.jit, static_argnames=())
def kernel(x, position_indices, embedding):
    b, s, d = x.shape
    n = b * s
    x_flat = x.reshape(n, d)
    idx_2d = position_indices.reshape(n // CHUNK, CHUNK).astype(jnp.int32)

    eb = lax.bitcast_convert_type(
        embedding.astype(jnp.bfloat16), jnp.uint16
    ).astype(jnp.uint32)
    half = d // 2
    packed = jnp.bitwise_or(
        eb[:, :half], jnp.left_shift(eb[:, half:], jnp.uint32(16))
    ).astype(jnp.int32)
    tab_flat = packed.reshape(-1)

    mesh = plsc.VectorSubcoreMesh(
        core_axis_name="c", subcore_axis_name="s", num_cores=2, num_subcores=16
    )
    n_chunks_w = n // NUM_WORKERS // CHUNK
    out = pl.kernel(
        _pos_embed_body,
        out_type=jax.ShapeDtypeStruct((n, d), x.dtype),
        mesh=mesh,
        scratch_types=[
            pltpu.VMEM(tab_flat.shape, jnp.int32),
            pltpu.VMEM((n_chunks_w, CHUNK), jnp.int32),
            pltpu.VMEM((NBUF, CHUNK, d), jnp.float32),
            pltpu.SemaphoreType.DMA((NBUF,)),
            pltpu.SemaphoreType.DMA((NBUF,)),
        ],
        compiler_params=pltpu.CompilerParams(
            use_tc_tiling_on_sc=False, needs_layout_passes=False
        ),
    )(x_flat, idx_2d, tab_flat)
    return out.reshape(b, s, d)


# speedup vs baseline: 2.1669x; 1.9332x over previous
import functools

import jax
import jax.numpy as jnp
from jax import lax
from jax.experimental import pallas as pl
from jax.experimental.pallas import tpu as pltpu
from jax.experimental.pallas import tpu_sc as plsc

NUM_WORKERS = 32
CHUNK = 128
NBUF = 4
LANES = 16


def _pos_embed_body(x_hbm, idx_hbm, tab_hbm, out_hbm, tb, idx_v, xb, xsem, osem):
    nc = 2
    wid = lax.axis_index("s") * nc + lax.axis_index("c")
    tok_per_worker = x_hbm.shape[0] // NUM_WORKERS
    n_chunks = tok_per_worker // CHUNK
    worker_base = wid * tok_per_worker
    chunk_base = wid * n_chunks

    pltpu.sync_copy(tab_hbm, tb)
    pltpu.sync_copy(idx_hbm.at[pl.ds(chunk_base, n_chunks)], idx_v)

    viota = lax.iota(jnp.int32, LANES)

    def issue_in(g):
        b = lax.rem(g, NBUF)
        base = worker_base + g * CHUNK
        pltpu.async_copy(x_hbm.at[pl.ds(base, CHUNK)], xb.at[b], xsem.at[b])

    def compute_and_out(g):
        b = lax.rem(g, NBUF)
        base = worker_base + g * CHUNK
        pltpu.make_async_copy(
            x_hbm.at[pl.ds(base, CHUNK)], xb.at[b], xsem.at[b]
        ).wait()

        @plsc.parallel_loop(0, CHUNK // LANES)
        def _grp(grp):
            iv = idx_v[g, pl.ds(grp * LANES, LANES)]
            ivs = iv * jnp.int32(33)
            for j in range(LANES):
                t = grp * LANES + j
                a0 = jnp.take(ivs, jnp.full((LANES,), j, jnp.int32)) + viota
                w0 = plsc.load_gather(tb, [a0])
                plsc.addupdate(
                    xb.at[b, t, pl.ds(0, LANES)],
                    lax.bitcast_convert_type(
                        lax.shift_left(w0, jnp.int32(16)), jnp.float32
                    ),
                )
                plsc.addupdate(
                    xb.at[b, t, pl.ds(2 * LANES, LANES)],
                    lax.bitcast_convert_type(
                        lax.bitwise_and(w0, jnp.int32(-65536)), jnp.float32
                    ),
                )
                w1 = plsc.load_gather(tb, [a0 + jnp.int32(LANES)])
                plsc.addupdate(
                    xb.at[b, t, pl.ds(LANES, LANES)],
                    lax.bitcast_convert_type(
                        lax.shift_left(w1, jnp.int32(16)), jnp.float32
                    ),
                )
                plsc.addupdate(
                    xb.at[b, t, pl.ds(3 * LANES, LANES)],
                    lax.bitcast_convert_type(
                        lax.bitwise_and(w1, jnp.int32(-65536)), jnp.float32
                    ),
                )

        pltpu.async_copy(xb.at[b], out_hbm.at[pl.ds(base, CHUNK)], osem.at[b])

    def wait_out(g):
        b = lax.rem(g, NBUF)
        base = worker_base + g * CHUNK
        pltpu.make_async_copy(
            xb.at[b], out_hbm.at[pl.ds(base, CHUNK)], osem.at[b]
        ).wait()

    @pl.loop(0, n_chunks + 1)
    def _step(t):
        @pl.when(jnp.logical_and(t >= NBUF, t - NBUF < n_chunks))
        def _():
            wait_out(t - NBUF)

        @pl.when(t < n_chunks)
        def _():
            issue_in(t)

        @pl.when(t >= 1)
        def _():
            compute_and_out(t - 1)

    @pl.loop(max(0, n_chunks + 1 - NBUF), n_chunks)
    def _drain(g):
        wait_out(g)


@functools.partial(jax.jit, static_argnames=())
def kernel(x, position_indices, embedding):
    b, s, d = x.shape
    n = b * s
    x_flat = x.reshape(n, d)
    idx_2d = position_indices.reshape(n // CHUNK, CHUNK).astype(jnp.int32)

    eb = lax.bitcast_convert_type(
        embedding.astype(jnp.bfloat16), jnp.uint16
    ).astype(jnp.uint32)
    half = d // 2
    packed = jnp.bitwise_or(
        eb[:, :half], jnp.left_shift(eb[:, half:], jnp.uint32(16))
    ).astype(jnp.int32)
    packed = jnp.pad(packed, ((0, 0), (0, 1)))
    tab_flat = packed.reshape(-1)

    mesh = plsc.VectorSubcoreMesh(
        core_axis_name="c", subcore_axis_name="s", num_cores=2, num_subcores=16
    )
    n_chunks_w = n // NUM_WORKERS // CHUNK
    out = pl.kernel(
        _pos_embed_body,
        out_type=jax.ShapeDtypeStruct((n, d), x.dtype),
        mesh=mesh,
        scratch_types=[
            pltpu.VMEM(tab_flat.shape, jnp.int32),
            pltpu.VMEM((n_chunks_w, CHUNK), jnp.int32),
            pltpu.VMEM((NBUF, CHUNK, d), jnp.float32),
            pltpu.SemaphoreType.DMA((NBUF,)),
            pltpu.SemaphoreType.DMA((NBUF,)),
        ],
        compiler_params=pltpu.CompilerParams(
            use_tc_tiling_on_sc=False, needs_layout_passes=False
        ),
    )(x_flat, idx_2d, tab_flat)
    return out.reshape(b, s, d)


# speedup vs baseline: 2.1714x; 1.0021x over previous
import functools

import jax
import jax.numpy as jnp
from jax import lax
from jax.experimental import pallas as pl
from jax.experimental.pallas import tpu as pltpu
from jax.experimental.pallas import tpu_sc as plsc

NUM_WORKERS = 32
CHUNK = 128
NBUF = 4
LANES = 16


def _pos_embed_body(x_hbm, idx_hbm, tab_hbm, out_hbm, tb, idx_v, xb, xsem, osem):
    nc = 2
    wid = lax.axis_index("s") * nc + lax.axis_index("c")
    tok_per_worker = x_hbm.shape[0] // NUM_WORKERS
    n_chunks = tok_per_worker // CHUNK
    worker_base = wid * tok_per_worker
    chunk_base = wid * n_chunks

    pltpu.sync_copy(tab_hbm, tb)
    pltpu.sync_copy(idx_hbm.at[pl.ds(chunk_base, n_chunks)], idx_v)

    viota = lax.iota(jnp.int32, LANES)

    def issue_in(g):
        b = lax.rem(g, NBUF)
        base = worker_base + g * CHUNK
        pltpu.async_copy(x_hbm.at[pl.ds(base, CHUNK)], xb.at[b], xsem.at[b])

    def compute_and_out(g):
        b = lax.rem(g, NBUF)
        base = worker_base + g * CHUNK
        pltpu.make_async_copy(
            x_hbm.at[pl.ds(base, CHUNK)], xb.at[b], xsem.at[b]
        ).wait()

        @plsc.parallel_loop(0, CHUNK // LANES, unroll=2)
        def _grp(grp):
            iv = idx_v[g, pl.ds(grp * LANES, LANES)]
            ivs = iv * jnp.int32(33)
            for j in range(LANES):
                t = grp * LANES + j
                a0 = jnp.take(ivs, jnp.full((LANES,), j, jnp.int32)) + viota
                w0 = plsc.load_gather(tb, [a0])
                plsc.addupdate(
                    xb.at[b, t, pl.ds(0, LANES)],
                    lax.bitcast_convert_type(
                        lax.shift_left(w0, jnp.int32(16)), jnp.float32
                    ),
                )
                plsc.addupdate(
                    xb.at[b, t, pl.ds(2 * LANES, LANES)],
                    lax.bitcast_convert_type(
                        lax.bitwise_and(w0, jnp.int32(-65536)), jnp.float32
                    ),
                )
                w1 = plsc.load_gather(tb, [a0 + jnp.int32(LANES)])
                plsc.addupdate(
                    xb.at[b, t, pl.ds(LANES, LANES)],
                    lax.bitcast_convert_type(
                        lax.shift_left(w1, jnp.int32(16)), jnp.float32
                    ),
                )
                plsc.addupdate(
                    xb.at[b, t, pl.ds(3 * LANES, LANES)],
                    lax.bitcast_convert_type(
                        lax.bitwise_and(w1, jnp.int32(-65536)), jnp.float32
                    ),
                )

        pltpu.async_copy(xb.at[b], out_hbm.at[pl.ds(base, CHUNK)], osem.at[b])

    def wait_out(g):
        b = lax.rem(g, NBUF)
        base = worker_base + g * CHUNK
        pltpu.make_async_copy(
            xb.at[b], out_hbm.at[pl.ds(base, CHUNK)], osem.at[b]
        ).wait()

    @pl.loop(0, n_chunks + 1)
    def _step(t):
        @pl.when(jnp.logical_and(t >= NBUF, t - NBUF < n_chunks))
        def _():
            wait_out(t - NBUF)

        @pl.when(t < n_chunks)
        def _():
            issue_in(t)

        @pl.when(t >= 1)
        def _():
            compute_and_out(t - 1)

    @pl.loop(max(0, n_chunks + 1 - NBUF), n_chunks)
    def _drain(g):
        wait_out(g)


@functools.partial(jax.jit, static_argnames=())
def kernel(x, position_indices, embedding):
    b, s, d = x.shape
    n = b * s
    x_flat = x.reshape(n, d)
    idx_2d = position_indices.reshape(n // CHUNK, CHUNK).astype(jnp.int32)

    eb = lax.bitcast_convert_type(
        embedding.astype(jnp.bfloat16), jnp.uint16
    ).astype(jnp.uint32)
    half = d // 2
    packed = jnp.bitwise_or(
        eb[:, :half], jnp.left_shift(eb[:, half:], jnp.uint32(16))
    ).astype(jnp.int32)
    packed = jnp.pad(packed, ((0, 0), (0, 1)))
    tab_flat = packed.reshape(-1)

    mesh = plsc.VectorSubcoreMesh(
        core_axis_name="c", subcore_axis_name="s", num_cores=2, num_subcores=16
    )
    n_chunks_w = n // NUM_WORKERS // CHUNK
    out = pl.kernel(
        _pos_embed_body,
        out_type=jax.ShapeDtypeStruct((n, d), x.dtype),
        mesh=mesh,
        scratch_types=[
            pltpu.VMEM(tab_flat.shape, jnp.int32),
            pltpu.VMEM((n_chunks_w, CHUNK), jnp.int32),
            pltpu.VMEM((NBUF, CHUNK, d), jnp.float32),
            pltpu.SemaphoreType.DMA((NBUF,)),
            pltpu.SemaphoreType.DMA((NBUF,)),
        ],
        compiler_params=pltpu.CompilerParams(
            use_tc_tiling_on_sc=False, needs_layout_passes=False
        ),
    )(x_flat, idx_2d, tab_flat)
    return out.reshape(b, s, d)
